# baseline (speedup 1.0000x reference)
import functools

import jax
import jax.numpy as jnp
from jax import lax
from jax.experimental import pallas as pl
from jax.experimental.pallas import tpu as pltpu
from jax.experimental.pallas import tpu_sc as plsc

_N_NODES = 10000
_ROW = 128
_NUM_TOK = 16
_TOK = 8
_EMB_ROWS = 256
_EMB_DIM = 8
_NW = 32
_SPAN = 314
_STRIDE = -(-_N_NODES // _NW)
_HALF = _SPAN // 2
_HALFW = _HALF * _ROW
_SPANW = _SPAN * _ROW
_CODES = _SPAN * _NUM_TOK

_mesh = plsc.VectorSubcoreMesh(core_axis_name="c", subcore_axis_name="s")


@functools.partial(
    pl.kernel,
    out_type=jax.ShapeDtypeStruct((_N_NODES * _ROW,), jnp.float32),
    mesh=_mesh,
    compiler_params=pltpu.CompilerParams(
        needs_layout_passes=False, use_tc_tiling_on_sc=False,
        skip_device_barrier=True, disable_bounds_checks=True,
        disable_semaphore_checks=True),
    scratch_types=[
        pltpu.VMEM((_SPANW,), jnp.int32),
        pltpu.VMEM((2048,), jnp.float32),
        pltpu.VMEM((_CODES,), jnp.int32),
        pltpu.VMEM((_SPANW,), jnp.float32),
        pltpu.SemaphoreType.DMA,
        pltpu.SemaphoreType.DMA,
    ],
)
def _node_emb(x_hbm, emb_hbm, out_hbm, xv, embv, codesv, outv, isem, osem):
    wid = lax.axis_index("s") * 2 + lax.axis_index("c")
    start = jnp.minimum(wid * _STRIDE, _N_NODES - _SPAN)
    xbase = start * _ROW

    lanes = lax.iota(jnp.int32, 16)
    col_base = lanes * _TOK
    epat = lanes & 7
    pair_base = lanes >> 3

    in1 = pltpu.async_copy(x_hbm.at[pl.ds(xbase, _HALFW)], xv.at[pl.ds(0, _HALFW)], isem)
    in2 = pltpu.async_copy(
        x_hbm.at[pl.ds(xbase + _HALFW, _HALFW)],
        xv.at[pl.ds(_HALFW, _HALFW)], isem)
    pltpu.sync_copy(emb_hbm, embv)

    def pack_node(n):
        nbase = col_base + n * _ROW
        codes = plsc.load_gather(xv, [nbase])
        for b in range(1, _TOK):
            plane = plsc.load_gather(xv, [nbase + b])
            codes = codes + (plane << b)
        codesv[pl.ds(n * _NUM_TOK, _NUM_TOK)] = codes

    def expand_node(n):
        cbase = n * _NUM_TOK + pair_base
        cpairs = [plsc.load_gather(codesv, [cbase + 2 * v])
                  for v in range(_ROW // 16)]
        vals = [plsc.load_gather(embv, [(c << 3) + epat]) for c in cpairs]
        for v, val in enumerate(vals):
            outv[pl.ds(n * _ROW + v * 16, 16)] = val

    in1.wait()
    plsc.parallel_loop(0, _HALF, unroll=8)(pack_node)
    plsc.parallel_loop(0, _HALF, unroll=16)(expand_node)
    out1 = pltpu.async_copy(
        outv.at[pl.ds(0, _HALFW)], out_hbm.at[pl.ds(xbase, _HALFW)], osem)

    in2.wait()
    plsc.parallel_loop(_HALF, _SPAN, unroll=8)(pack_node)
    plsc.parallel_loop(_HALF, _SPAN, unroll=16)(expand_node)
    pltpu.sync_copy(
        outv.at[pl.ds(_HALFW, _HALFW)],
        out_hbm.at[pl.ds(xbase + _HALFW, _HALFW)])
    out1.wait()


def kernel(x, edge_index, emb_table):
    node_flat = _node_emb(
        x.reshape(-1).astype(jnp.int32), emb_table.reshape(-1))
    node_vec = node_flat.reshape(_N_NODES, _ROW)
    edge_vec = jnp.zeros((edge_index.shape[-1], _ROW), dtype=jnp.float32)
    return (node_vec, edge_index, edge_vec)


# speedup vs baseline: 1.1573x; 1.1573x over previous
import functools

import jax
import jax.numpy as jnp
from jax import lax
from jax.experimental import pallas as pl
from jax.experimental.pallas import tpu as pltpu
from jax.experimental.pallas import tpu_sc as plsc

_N_NODES = 10000
_ROW = 128
_NUM_TOK = 16
_TOK = 8
_EMB_ROWS = 256
_EMB_DIM = 8
_NW = 32
_SPAN = 314
_STRIDE = -(-_N_NODES // _NW)
_HALF = _SPAN // 2
_HALFW = _HALF * _ROW
_SPANW = _SPAN * _ROW
_CODES = _SPAN * _NUM_TOK

_mesh = plsc.VectorSubcoreMesh(core_axis_name="c", subcore_axis_name="s")


@functools.partial(
    pl.kernel,
    out_type=jax.ShapeDtypeStruct((_N_NODES * _ROW,), jnp.float32),
    mesh=_mesh,
    compiler_params=pltpu.CompilerParams(
        needs_layout_passes=False, use_tc_tiling_on_sc=False,
        skip_device_barrier=True, disable_bounds_checks=True,
        disable_semaphore_checks=True),
    scratch_types=[
        pltpu.VMEM((_SPANW,), jnp.int32),
        pltpu.VMEM((2048,), jnp.float32),
        pltpu.VMEM((_CODES,), jnp.int32),
        pltpu.VMEM((_SPANW,), jnp.float32),
        pltpu.SemaphoreType.DMA,
        pltpu.SemaphoreType.DMA,
    ],
)
def _node_emb(x_hbm, emb_hbm, out_hbm, xv, embv, codesv, outv, isem, osem):
    wid = lax.axis_index("s") * 2 + lax.axis_index("c")
    start = jnp.minimum(wid * _STRIDE, _N_NODES - _SPAN)
    xbase = start * _ROW

    lanes = lax.iota(jnp.int32, 16)
    col_base = lanes * _TOK
    epat = lanes & 7
    pair_base = lanes >> 3

    in1 = pltpu.async_copy(x_hbm.at[pl.ds(xbase, _HALFW)], xv.at[pl.ds(0, _HALFW)], isem)
    in2 = pltpu.async_copy(
        x_hbm.at[pl.ds(xbase + _HALFW, _HALFW)],
        xv.at[pl.ds(_HALFW, _HALFW)], isem)
    pltpu.sync_copy(emb_hbm, embv)

    def pack_node(n):
        nbase = col_base + n * _ROW
        codes = plsc.load_gather(xv, [nbase])
        for b in range(1, _TOK):
            plane = plsc.load_gather(xv, [nbase + b])
            codes = codes + (plane << b)
        codesv[pl.ds(n * _NUM_TOK, _NUM_TOK)] = codes

    def expand_node(n):
        cbase = n * _NUM_TOK + pair_base
        cpairs = [plsc.load_gather(codesv, [cbase + 2 * v])
                  for v in range(_ROW // 16)]
        vals = [plsc.load_gather(embv, [(c << 3) + epat]) for c in cpairs]
        for v, val in enumerate(vals):
            outv[pl.ds(n * _ROW + v * 16, 16)] = val

    in1.wait()
    plsc.parallel_loop(0, _HALF, unroll=4)(pack_node)
    plsc.parallel_loop(0, _HALF, unroll=4)(expand_node)
    out1 = pltpu.async_copy(
        outv.at[pl.ds(0, _HALFW)], out_hbm.at[pl.ds(xbase, _HALFW)], osem)

    in2.wait()
    plsc.parallel_loop(_HALF, _SPAN, unroll=4)(pack_node)
    plsc.parallel_loop(_HALF, _SPAN, unroll=4)(expand_node)
    pltpu.sync_copy(
        outv.at[pl.ds(_HALFW, _HALFW)],
        out_hbm.at[pl.ds(xbase + _HALFW, _HALFW)])
    out1.wait()


def kernel(x, edge_index, emb_table):
    node_flat = _node_emb(
        x.reshape(-1).astype(jnp.int32), emb_table.reshape(-1))
    node_vec = node_flat.reshape(_N_NODES, _ROW)
    edge_vec = jnp.zeros((edge_index.shape[-1], _ROW), dtype=jnp.float32)
    return (node_vec, edge_index, edge_vec)
